# Initial kernel scaffold; baseline (speedup 1.0000x reference)
import jax
import jax.numpy as jnp
from jax.experimental import pallas as pl


def kernel(x, edge_attr, edge_index, batch, enc_W, enc_b, eenc_W, eenc_b, mlp1_W, mlp1_b, mlp_ln_g, mlp_ln_b, mlp2_W, mlp2_b, t, ln_g, ln_b):
    raise NotImplementedError("write your pallas kernel here")


# speedup vs baseline: 5.0682x; 5.0682x over previous
import functools

import jax
import jax.numpy as jnp
from jax import lax
from jax.experimental import pallas as pl
from jax.experimental.pallas import tpu as pltpu
from jax.experimental.pallas import tpu_sc as plsc

N = 10000
E = 320000
H = 128
HH = H // 2
L = 4
G = 128
EPS = 1e-7

NCORE = 2
NSUB = 16
LANE = 16

EDGES_PER_TILE = E // NSUB
CHUNK = 200
NCHUNK = EDGES_PER_TILE // CHUNK
ROWS_PER_TILE = 640

_F32 = jnp.float32


def _enc_body(x_ref, w_ref, b_ref, h_ref, hs_ref):
    h = jnp.dot(x_ref[...], w_ref[...], preferred_element_type=_F32) + b_ref[...]
    h_ref[...] = h
    hs_ref[0] = h[:, :HH]
    hs_ref[1] = h[:, HH:]


def _enc_call(x, w, b, rows, bn):
    grid = rows // bn
    return pl.pallas_call(
        _enc_body,
        grid=(grid,),
        in_specs=[
            pl.BlockSpec((bn, x.shape[1]), lambda n: (n, 0)),
            pl.BlockSpec(w.shape, lambda n: (0, 0)),
            pl.BlockSpec((1, H), lambda n: (0, 0)),
        ],
        out_specs=[
            pl.BlockSpec((bn, H), lambda n: (n, 0)),
            pl.BlockSpec((2, bn, HH), lambda n: (0, n, 0)),
        ],
        out_shape=[
            jax.ShapeDtypeStruct((rows, H), _F32),
            jax.ShapeDtypeStruct((2, rows, HH), _F32),
        ],
    )(x, w, b)


def _mlp_body(agg_ref, r_ref, res_ref, w1_ref, b1_ref, g1_ref, bb1_ref,
              w2_ref, b2_ref, gn_ref, bn_ref, h_ref, rn_ref, rns_ref):
    u = jnp.concatenate([agg_ref[0], agg_ref[1]], axis=1) + r_ref[...]
    z = jnp.dot(u, w1_ref[...], preferred_element_type=_F32) + b1_ref[...]
    mu = jnp.mean(z, axis=-1, keepdims=True)
    var = jnp.mean((z - mu) ** 2, axis=-1, keepdims=True)
    z = (z - mu) * lax.rsqrt(var + 1e-5) * g1_ref[...] + bb1_ref[...]
    z = jnp.maximum(z, 0.0)
    y = jnp.dot(z, w2_ref[...], preferred_element_type=_F32) + b2_ref[...]
    y = y + res_ref[...]
    h_ref[...] = y
    mu2 = jnp.mean(y, axis=-1, keepdims=True)
    var2 = jnp.mean((y - mu2) ** 2, axis=-1, keepdims=True)
    rn = jnp.maximum((y - mu2) * lax.rsqrt(var2 + 1e-5) * gn_ref[...] + bn_ref[...], 0.0)
    rn_ref[...] = rn
    rns_ref[0] = rn[:, :HH]
    rns_ref[1] = rn[:, HH:]


def _mlp_call(agg, r, res, w1, b1, g1, bb1, w2, b2, gn, bnb):
    bn = 1000
    grid = N // bn
    return pl.pallas_call(
        _mlp_body,
        grid=(grid,),
        in_specs=[
            pl.BlockSpec((2, bn, HH), lambda n: (0, n, 0)),
            pl.BlockSpec((bn, H), lambda n: (n, 0)),
            pl.BlockSpec((bn, H), lambda n: (n, 0)),
            pl.BlockSpec((H, 2 * H), lambda n: (0, 0)),
            pl.BlockSpec((1, 2 * H), lambda n: (0, 0)),
            pl.BlockSpec((1, 2 * H), lambda n: (0, 0)),
            pl.BlockSpec((1, 2 * H), lambda n: (0, 0)),
            pl.BlockSpec((2 * H, H), lambda n: (0, 0)),
            pl.BlockSpec((1, H), lambda n: (0, 0)),
            pl.BlockSpec((1, H), lambda n: (0, 0)),
            pl.BlockSpec((1, H), lambda n: (0, 0)),
        ],
        out_specs=[
            pl.BlockSpec((bn, H), lambda n: (n, 0)),
            pl.BlockSpec((bn, H), lambda n: (n, 0)),
            pl.BlockSpec((2, bn, HH), lambda n: (0, n, 0)),
        ],
        out_shape=[
            jax.ShapeDtypeStruct((N, H), _F32),
            jax.ShapeDtypeStruct((N, H), _F32),
            jax.ShapeDtypeStruct((2, N, HH), _F32),
        ],
    )(agg, r, res, w1, b1, g1, bb1, w2, b2, gn, bnb)


def _pool_body(h_ref, b_ref, o_ref, acc, cnt):
    step = pl.program_id(0)

    @pl.when(step == 0)
    def _():
        acc[...] = jnp.zeros_like(acc)
        cnt[...] = jnp.zeros_like(cnt)

    onehot = (b_ref[...] == lax.broadcasted_iota(jnp.int32, (1, G), 1)).astype(_F32)
    acc[...] += lax.dot_general(onehot, h_ref[...], (((0,), (0,)), ((), ())),
                                preferred_element_type=_F32)
    cnt[...] += lax.dot_general(onehot, jnp.ones_like(h_ref[...]),
                                (((0,), (0,)), ((), ())), preferred_element_type=_F32)

    @pl.when(step == pl.num_programs(0) - 1)
    def _():
        o_ref[...] = acc[...] / jnp.maximum(cnt[...], 1.0)


def _pool_call(h, batch2d):
    bn = 1000
    grid = N // bn
    return pl.pallas_call(
        _pool_body,
        grid=(grid,),
        in_specs=[
            pl.BlockSpec((bn, H), lambda n: (n, 0)),
            pl.BlockSpec((bn, 1), lambda n: (n, 0)),
        ],
        out_specs=pl.BlockSpec((G, H), lambda n: (0, 0)),
        out_shape=jax.ShapeDtypeStruct((G, H), _F32),
        scratch_shapes=[
            pltpu.VMEM((G, H), _F32),
            pltpu.VMEM((G, H), _F32),
        ],
    )(h, batch2d)


def _agg_body(r_hbm, ea_hbm, src_hbm, dst_hbm, t_hbm, agg_hbm,
              idx_v, dst_v, hrow_v, ea_v, t_v, num_s, den_s, sem):
    c = lax.axis_index("c")
    s = lax.axis_index("s")
    coff = c * N

    def zbody(e, _):
        for j in range(HH // LANE):
            hrow_v[e, pl.ds(j * LANE, LANE)] = jnp.zeros((LANE,), _F32)
        return 0

    lax.fori_loop(0, CHUNK, zbody, 0)
    row0 = pl.multiple_of(jnp.minimum(s * ROWS_PER_TILE, N - ROWS_PER_TILE), 8)
    pltpu.sync_copy(hrow_v, num_s.at[pl.ds(row0, CHUNK)])
    pltpu.sync_copy(hrow_v, den_s.at[pl.ds(row0, CHUNK)])
    rem = ROWS_PER_TILE - CHUNK
    pltpu.sync_copy(hrow_v.at[pl.ds(0, rem)], num_s.at[pl.ds(row0 + CHUNK, rem)])
    pltpu.sync_copy(hrow_v.at[pl.ds(0, rem)], den_s.at[pl.ds(row0 + CHUNK, rem)])
    plsc.subcore_barrier()

    pltpu.sync_copy(t_hbm, t_v)
    tv = t_v[...]
    ebase = s * EDGES_PER_TILE

    def chunk_body(ci, _):
        eoff = pl.multiple_of(ebase + ci * CHUNK, 8)
        pltpu.sync_copy(src_hbm.at[pl.ds(eoff, CHUNK)], idx_v)
        pltpu.sync_copy(dst_hbm.at[pl.ds(eoff, CHUNK)], dst_v)

        def adj(k, _):
            sl = pl.ds(k * LANE, LANE)
            idx_v[sl] = idx_v[sl] + coff
            return 0

        lax.fori_loop(0, CHUNK // LANE, adj, 0)
        pltpu.async_copy(r_hbm.at[idx_v], hrow_v, sem).wait()
        pltpu.sync_copy(ea_hbm.at[pl.ds(pl.multiple_of(c * E + eoff, 8), CHUNK)], ea_v)

        def ebody(e, _):
            for j in range(HH // LANE):
                sl = pl.ds(j * LANE, LANE)
                m = jnp.maximum(hrow_v[e, sl] + ea_v[e, sl], 0.0) + EPS
                w = jnp.exp(tv * m)
                ea_v[e, sl] = w
                hrow_v[e, sl] = m * w
            return 0

        lax.fori_loop(0, CHUNK, ebody, 0)
        pltpu.sync_copy(ea_v, den_s.at[dst_v], add=True)
        pltpu.sync_copy(hrow_v, num_s.at[dst_v], add=True)
        return 0

    lax.fori_loop(0, NCHUNK, chunk_body, 0)
    plsc.subcore_barrier()

    for off, sz in ((0, CHUNK), (CHUNK, ROWS_PER_TILE - CHUNK)):
        pltpu.sync_copy(num_s.at[pl.ds(row0 + off, sz)], hrow_v.at[pl.ds(0, sz)])
        pltpu.sync_copy(den_s.at[pl.ds(row0 + off, sz)], ea_v.at[pl.ds(0, sz)])

        def fbody(e, _):
            for j in range(HH // LANE):
                sl = pl.ds(j * LANE, LANE)
                hrow_v[e, sl] = hrow_v[e, sl] / (ea_v[e, sl] + 1e-16)
            return 0

        lax.fori_loop(0, sz, fbody, 0)
        pltpu.sync_copy(hrow_v.at[pl.ds(0, sz)],
                        agg_hbm.at[pl.ds(pl.multiple_of(coff + row0 + off, 8), sz)])


@functools.lru_cache(maxsize=1)
def _agg_kernel():
    return pl.kernel(
        _agg_body,
        out_type=jax.ShapeDtypeStruct((2 * N, HH), _F32),
        mesh=plsc.VectorSubcoreMesh(core_axis_name="c", subcore_axis_name="s",
                                    num_cores=NCORE, num_subcores=NSUB),
        compiler_params=pltpu.CompilerParams(use_tc_tiling_on_sc=False),
        scratch_types=[
            pltpu.VMEM((CHUNK,), jnp.int32),
            pltpu.VMEM((CHUNK,), jnp.int32),
            pltpu.VMEM((CHUNK, HH), _F32),
            pltpu.VMEM((CHUNK, HH), _F32),
            pltpu.VMEM((LANE,), _F32),
            pltpu.VMEM_SHARED((N, HH), _F32),
            pltpu.VMEM_SHARED((N, HH), _F32),
            pltpu.SemaphoreType.DMA,
        ],
    )


def _agg_call(r_hbm, ea_hbm, src_hbm, dst_hbm, t_hbm):
    return _agg_kernel()(r_hbm, ea_hbm, src_hbm, dst_hbm, t_hbm)


def kernel(x, edge_attr, edge_index, batch, enc_W, enc_b, eenc_W, eenc_b,
           mlp1_W, mlp1_b, mlp_ln_g, mlp_ln_b, mlp2_W, mlp2_b, t, ln_g, ln_b):
    src = edge_index[0]
    dst = edge_index[1]

    h0, h0s = _enc_call(x, enc_W, enc_b.reshape(1, H), N, 1000)
    _, eas = _enc_call(edge_attr, eenc_W, eenc_b.reshape(1, H), E, 4000)
    ea_flat = eas.reshape(2 * E, HH)

    r = h0
    rs = h0s.reshape(2 * N, HH)
    res = jnp.zeros((N, H), _F32)
    rn = r
    for i in range(L):
        t16 = jnp.full((LANE,), t[i], _F32)
        agg = _agg_call(rs, ea_flat, src, dst, t16)
        gi = 0 if i == L - 1 else i + 1
        h, rn, rns = _mlp_call(
            agg.reshape(2, N, HH), r, res,
            mlp1_W[i], mlp1_b[i].reshape(1, 2 * H),
            mlp_ln_g[i].reshape(1, 2 * H), mlp_ln_b[i].reshape(1, 2 * H),
            mlp2_W[i], mlp2_b[i].reshape(1, H),
            ln_g[gi].reshape(1, H), ln_b[gi].reshape(1, H))
        res = h
        r = rn
        rs = rns.reshape(2 * N, HH)

    return _pool_call(rn, batch.reshape(N, 1))
